# baseline (speedup 1.0000x reference)
import functools

import jax
import jax.numpy as jnp
from jax import lax
from jax.experimental import pallas as pl
from jax.experimental.pallas import tpu as pltpu
from jax.experimental.pallas import tpu_sc as plsc

NC = 2
NS = 16
NW = NC * NS
LANES = 16


def _make_sc_kernel(B, C, E):
    b_per_w = B // NW
    CB = 128
    n_chunks = b_per_w // CB
    ctx_per_chunk = CB * C
    n_ctx_g = ctx_per_chunk // 128
    ne = E // LANES

    mesh = plsc.VectorSubcoreMesh(core_axis_name="c", subcore_axis_name="s")

    @functools.partial(
        pl.kernel,
        mesh=mesh,
        compiler_params=pltpu.CompilerParams(needs_layout_passes=False,
                                             use_tc_tiling_on_sc=False),
        out_type=jax.ShapeDtypeStruct((B * C,), jnp.float32),
        scratch_types=[
            pltpu.VMEM((b_per_w,), jnp.int32),
            pltpu.VMEM((b_per_w * C,), jnp.int32),
            pltpu.VMEM((CB, E), jnp.float32),
            pltpu.VMEM((ctx_per_chunk, E), jnp.float32),
            pltpu.VMEM((ctx_per_chunk, LANES), jnp.float32),
            pltpu.VMEM((b_per_w * C,), jnp.float32),
            pltpu.SemaphoreType.DMA,
        ],
    )
    def sc_kernel(tgt_hbm, ctx_hbm, ttab_hbm, ctab_hbm, out_hbm,
                  tgt_idx_v, ctx_idx_v, tgt_rows_v, ctx_rows_v, buf_v,
                  out_v, sem):
        wid = lax.axis_index("s") * NC + lax.axis_index("c")
        base = wid * b_per_w
        pltpu.sync_copy(tgt_hbm.at[pl.ds(base, b_per_w)], tgt_idx_v)
        pltpu.sync_copy(ctx_hbm.at[pl.ds(base * C, b_per_w * C)], ctx_idx_v)

        lanes = lax.iota(jnp.int32, LANES)
        last = jnp.full((LANES,), LANES - 1, jnp.int32)

        for k in range(n_chunks):
            cps = [pltpu.async_copy(
                ttab_hbm.at[tgt_idx_v.at[pl.ds(k * CB, CB)]], tgt_rows_v, sem)]
            for g in range(n_ctx_g):
                cps.append(pltpu.async_copy(
                    ctab_hbm.at[ctx_idx_v.at[pl.ds(k * ctx_per_chunk + g * 128,
                                                   128)]],
                    ctx_rows_v.at[pl.ds(g * 128, 128)], sem))
            for cp in cps:
                cp.wait()

            def body(i, carry):
                t = [tgt_rows_v[i, pl.ds(e * LANES, LANES)] for e in range(ne)]
                r = i * C
                for c in range(C):
                    p = t[0] * ctx_rows_v[r + c, pl.ds(0, LANES)]
                    for e in range(1, ne):
                        p = p + t[e] * ctx_rows_v[r + c,
                                                  pl.ds(e * LANES, LANES)]
                    buf_v[r + c, pl.ds(0, LANES)] = plsc.cumsum(p)
                return carry

            lax.fori_loop(0, CB, body, 0)

            def collect(t, carry, k=k):
                rows = t * LANES + lanes
                sums = plsc.load_gather(buf_v, [rows, last])
                out_v[pl.ds(k * ctx_per_chunk + t * LANES, LANES)] = sums
                return carry

            lax.fori_loop(0, ctx_per_chunk // LANES, collect, 0)

        pltpu.sync_copy(out_v, out_hbm.at[pl.ds(base * C, b_per_w * C)])

    return sc_kernel


def kernel(target, context, target_table, context_table):
    B, C = context.shape
    E = target_table.shape[1]
    sc = _make_sc_kernel(B, C, E)
    return sc(target, context.reshape(-1), target_table,
              context_table).reshape(B, C)
